# baseline (speedup 1.0000x reference)
import jax
import jax.numpy as jnp
from jax.experimental import pallas as pl
from jax.experimental.pallas import tpu as pltpu

_BLK = 512


def _pe_add_kernel(x_ref, pe_ref, o_ref):
    o_ref[...] = x_ref[...] + pe_ref[...][None, :, :]


def kernel(x, pe):
    B, L, E = x.shape
    pe_rows = jax.lax.slice(pe, (1, 0), (1 + L, E))
    return pl.pallas_call(
        _pe_add_kernel,
        grid=(L // _BLK,),
        in_specs=[
            pl.BlockSpec((B, _BLK, E), lambda j: (0, j, 0)),
            pl.BlockSpec((_BLK, E), lambda j: (j, 0)),
        ],
        out_specs=pl.BlockSpec((B, _BLK, E), lambda j: (0, j, 0)),
        out_shape=jax.ShapeDtypeStruct((B, L, E), x.dtype),
        compiler_params=pltpu.CompilerParams(
            dimension_semantics=("parallel",),
        ),
    )(x, pe_rows)
